# baseline (speedup 1.0000x reference)
import functools
import math

import jax
import jax.numpy as jnp
from jax.experimental import pallas as pl
from jax.experimental.pallas import tpu as pltpu

_INV_SQRT2 = 1.0 / math.sqrt(2.0)


def _body(x_ref, w1_ref, b1_ref, w2_ref, b2_ref, o_ref, pooled_ref,
          *, nb, nh, hblk, inv_hw):
    i = pl.program_id(0)
    j = pl.program_id(1)
    s = x_ref[:, 0:8]
    for t in range(1, hblk // 8):
        s = s + x_ref[:, 8 * t:8 * t + 8]
    r = jnp.sum(s, axis=(1, 2))

    @pl.when(j == 0)
    def _first():
        pooled_ref[pl.ds(i, 1), :] = r

    @pl.when(j != 0)
    def _acc():
        pooled_ref[pl.ds(i, 1), :] += r

    @pl.when((i == nb - 1) & (j == nh - 1))
    def _finish():
        p = pooled_ref[...] * inv_hw
        hpre = jnp.dot(p, w1_ref[...],
                       preferred_element_type=jnp.float32,
                       precision=jax.lax.Precision.HIGHEST) + b1_ref[...]
        hact = 0.5 * hpre * (1.0 + jax.lax.erf(hpre * _INV_SQRT2))
        logits = jnp.dot(hact, w2_ref[...],
                         preferred_element_type=jnp.float32,
                         precision=jax.lax.Precision.HIGHEST) + b2_ref[...]
        mx = jnp.max(logits, axis=-1, keepdims=True)
        e = jnp.exp(logits - mx)
        o_ref[...] = e / jnp.sum(e, axis=-1, keepdims=True)


@jax.jit
def kernel(img_emb, W1, b1, W2, b2):
    B, C, H, W = img_emb.shape
    HID = W1.shape[1]
    E = W2.shape[1]
    inv_hw = 1.0 / (H * W)

    xt = jnp.transpose(img_emb, (0, 2, 3, 1))

    HBLK = 32
    NH = H // HBLK
    out = pl.pallas_call(
        functools.partial(_body, nb=B, nh=NH, hblk=HBLK, inv_hw=inv_hw),
        grid=(B, NH),
        in_specs=[
            pl.BlockSpec((1, HBLK, W, C), lambda i, j: (i, j, 0, 0)),
            pl.BlockSpec((C, HID), lambda i, j: (0, 0)),
            pl.BlockSpec((1, HID), lambda i, j: (0, 0)),
            pl.BlockSpec((HID, E), lambda i, j: (0, 0)),
            pl.BlockSpec((1, E), lambda i, j: (0, 0)),
        ],
        out_specs=pl.BlockSpec((B, E), lambda i, j: (0, 0)),
        out_shape=jax.ShapeDtypeStruct((B, E), jnp.float32),
        scratch_shapes=[pltpu.VMEM((B, C), jnp.float32)],
    )(xt, W1, b1.reshape(1, -1), W2, b2.reshape(1, -1))
    return out


# speedup vs baseline: 1.3438x; 1.3438x over previous
import functools
import math

import jax
import jax.numpy as jnp
from jax.experimental import pallas as pl
from jax.experimental.pallas import tpu as pltpu

_INV_SQRT2 = 1.0 / math.sqrt(2.0)


def _body(xa_ref, xb_ref, w1_ref, b1_ref, w2_ref, b2_ref, o_ref,
          pooled_ref, *, nb, hblk, inv_hw):
    i = pl.program_id(0)
    s = xa_ref[:, 0:8] + xb_ref[:, 0:8]
    for t in range(1, hblk // 8):
        s = s + xa_ref[:, 8 * t:8 * t + 8]
        s = s + xb_ref[:, 8 * t:8 * t + 8]
    pooled_ref[pl.ds(i, 1), :] = jnp.sum(s, axis=(1, 2))

    @pl.when(i == nb - 1)
    def _finish():
        p = pooled_ref[...] * inv_hw
        hpre = jnp.dot(p, w1_ref[...],
                       preferred_element_type=jnp.float32,
                       precision=jax.lax.Precision.HIGHEST) + b1_ref[...]
        hact = 0.5 * hpre * (1.0 + jax.lax.erf(hpre * _INV_SQRT2))
        logits = jnp.dot(hact, w2_ref[...],
                         preferred_element_type=jnp.float32,
                         precision=jax.lax.Precision.HIGHEST) + b2_ref[...]
        mx = jnp.max(logits, axis=-1, keepdims=True)
        e = jnp.exp(logits - mx)
        o_ref[...] = e / jnp.sum(e, axis=-1, keepdims=True)


@jax.jit
def kernel(img_emb, W1, b1, W2, b2):
    B, C, H, W = img_emb.shape
    HID = W1.shape[1]
    E = W2.shape[1]
    inv_hw = 1.0 / (H * W)

    xt = jnp.transpose(img_emb, (0, 2, 3, 1))

    HBLK = H // 2
    out = pl.pallas_call(
        functools.partial(_body, nb=B, hblk=HBLK, inv_hw=inv_hw),
        grid=(B,),
        in_specs=[
            pl.BlockSpec((1, HBLK, W, C), lambda i: (i, 0, 0, 0)),
            pl.BlockSpec((1, HBLK, W, C), lambda i: (i, 1, 0, 0)),
            pl.BlockSpec((C, HID), lambda i: (0, 0)),
            pl.BlockSpec((1, HID), lambda i: (0, 0)),
            pl.BlockSpec((HID, E), lambda i: (0, 0)),
            pl.BlockSpec((1, E), lambda i: (0, 0)),
        ],
        out_specs=pl.BlockSpec((B, E), lambda i: (0, 0)),
        out_shape=jax.ShapeDtypeStruct((B, E), jnp.float32),
        scratch_shapes=[pltpu.VMEM((B, C), jnp.float32)],
    )(xt, xt, W1, b1.reshape(1, -1), W2, b2.reshape(1, -1))
    return out
